# baseline (speedup 1.0000x reference)
import functools

import jax
import jax.numpy as jnp
from jax.experimental import pallas as pl

_BUMP_STEPS = 32
_RADIUS = 1.1
_SCALE = 100.0
_NUM_SEGMENTS = 16
_NUM_THETAS = 32
_CHUNK = 2000


def _ect_kernel(x_ref, b_ref, v_ref, k_ref, c_ref, out_ref):
    i = pl.program_id(0)

    @pl.when(i == 0)
    def _init():
        out_ref[...] = jnp.zeros_like(out_ref)

    nh = jnp.dot(x_ref[...], v_ref[...], preferred_element_type=jnp.float32)
    half = _BUMP_STEPS // 2 * _NUM_THETAS
    parts = []
    for j in range(2):
        c = c_ref[0, j]
        e = jnp.exp(jnp.clip(nh - c, -88.0, 88.0))
        et = jnp.tile(e, (1, _BUMP_STEPS // 2))
        m = et * k_ref[0:1, j * half:(j + 1) * half]
        parts.append(1.0 / (1.0 + m))
    ecc = jnp.concatenate(parts, axis=1)
    seg = b_ref[...]
    oh = (seg == jax.lax.broadcasted_iota(jnp.int32, (1, _NUM_SEGMENTS), 1)
          ).astype(jnp.float32)
    partial = jax.lax.dot_general(
        oh, ecc, (((0,), (0,)), ((), ())),
        preferred_element_type=jnp.float32)
    out_ref[...] += partial


@jax.jit
def kernel(x, batch, v):
    n = x.shape[0]
    grid = n // _CHUNK
    st = _BUMP_STEPS * _NUM_THETAS
    lin = jnp.linspace(-_RADIUS, _RADIUS, _BUMP_STEPS, dtype=jnp.float32)
    a = _SCALE * lin
    half = _BUMP_STEPS // 2
    c0 = (a[0] + a[half - 1]) / 2.0
    c1 = (a[half] + a[-1]) / 2.0
    centers = jnp.stack([c0, c1]).reshape(1, 2)
    c_per_s = jnp.where(jnp.arange(_BUMP_STEPS) < half, c0, c1)
    k_row = jnp.exp(jnp.repeat(c_per_s - a, _NUM_THETAS)).reshape(1, st)
    k_row = k_row.astype(jnp.float32)
    v_scaled = (v * _SCALE).astype(jnp.float32)
    batch2d = batch.reshape(n, 1)
    out = pl.pallas_call(
        _ect_kernel,
        grid=(grid,),
        in_specs=[
            pl.BlockSpec((_CHUNK, x.shape[1]), lambda i: (i, 0)),
            pl.BlockSpec((_CHUNK, 1), lambda i: (i, 0)),
            pl.BlockSpec((v.shape[0], _NUM_THETAS), lambda i: (0, 0)),
            pl.BlockSpec((1, st), lambda i: (0, 0)),
            pl.BlockSpec((1, 2), lambda i: (0, 0)),
        ],
        out_specs=pl.BlockSpec((_NUM_SEGMENTS, st), lambda i: (0, 0)),
        out_shape=jax.ShapeDtypeStruct((_NUM_SEGMENTS, st), jnp.float32),
    )(x, batch2d, v_scaled, k_row, centers)
    return out.reshape(_NUM_SEGMENTS, _BUMP_STEPS, _NUM_THETAS)


# speedup vs baseline: 2.1839x; 2.1839x over previous
import numpy as np

import jax
import jax.numpy as jnp
from jax.experimental import pallas as pl

_BUMP_STEPS = 32
_RADIUS = 1.1
_SCALE = 100.0
_NUM_SEGMENTS = 16
_NUM_THETAS = 32
_CHUNK = 2048
_N_PAD = 51200

_lin64 = np.linspace(-_RADIUS, _RADIUS, _BUMP_STEPS)
_a64 = _SCALE * _lin64
_H = _BUMP_STEPS // 2
_C0 = float((_a64[0] + _a64[_H - 1]) / 2.0)
_C1 = float((_a64[_H] + _a64[-1]) / 2.0)
_KCONST = [float(np.exp((_C0 if s < _H else _C1) - _a64[s]))
           for s in range(_BUMP_STEPS)]


def _ect_kernel(xt_ref, b_ref, vt_ref, out_ref):
    i = pl.program_id(0)

    @pl.when(i == 0)
    def _init():
        out_ref[...] = jnp.zeros_like(out_ref)

    nh = jnp.dot(vt_ref[...], xt_ref[...],
                 preferred_element_type=jnp.float32)
    e0 = jnp.exp(jnp.clip(nh - _C0, -88.0, 88.0)).astype(jnp.bfloat16)
    e1 = jnp.exp(jnp.clip(nh - _C1, -88.0, 88.0)).astype(jnp.bfloat16)
    one = jnp.bfloat16(1.0)
    blocks = []
    for s in range(_BUMP_STEPS):
        e = e0 if s < _H else e1
        m1 = one + e * jnp.bfloat16(_KCONST[s])
        blocks.append(one / m1)
    sig = jnp.concatenate(blocks, axis=0)
    seg = b_ref[...]
    oh = (seg == jax.lax.broadcasted_iota(jnp.int32, (1, _NUM_SEGMENTS), 1)
          ).astype(jnp.bfloat16)
    out_ref[...] += jnp.dot(sig, oh,
                            preferred_element_type=jnp.float32)


@jax.jit
def kernel(x, batch, v):
    n = x.shape[0]
    st = _BUMP_STEPS * _NUM_THETAS
    grid = _N_PAD // _CHUNK
    xt = jnp.pad(x, ((0, _N_PAD - n), (0, 0))).T
    batch2d = jnp.pad(batch, (0, _N_PAD - n),
                      constant_values=_NUM_SEGMENTS).reshape(_N_PAD, 1)
    vt = (_SCALE * v.T).astype(jnp.float32)
    out = pl.pallas_call(
        _ect_kernel,
        grid=(grid,),
        in_specs=[
            pl.BlockSpec((x.shape[1], _CHUNK), lambda i: (0, i)),
            pl.BlockSpec((_CHUNK, 1), lambda i: (i, 0)),
            pl.BlockSpec((_NUM_THETAS, v.shape[0]), lambda i: (0, 0)),
        ],
        out_specs=pl.BlockSpec((st, _NUM_SEGMENTS), lambda i: (0, 0)),
        out_shape=jax.ShapeDtypeStruct((st, _NUM_SEGMENTS), jnp.float32),
    )(xt, batch2d, vt)
    return out.T.reshape(_NUM_SEGMENTS, _BUMP_STEPS, _NUM_THETAS)


# speedup vs baseline: 2.2185x; 1.0158x over previous
import numpy as np

import jax
import jax.numpy as jnp
from jax.experimental import pallas as pl

_BUMP_STEPS = 32
_RADIUS = 1.1
_SCALE = 100.0
_NUM_SEGMENTS = 16
_NUM_THETAS = 32
_CHUNK = 4096
_N_PAD = 53248

_lin64 = np.linspace(-_RADIUS, _RADIUS, _BUMP_STEPS)
_a64 = _SCALE * _lin64
_H = _BUMP_STEPS // 2
_C0 = float((_a64[0] + _a64[_H - 1]) / 2.0)
_C1 = float((_a64[_H] + _a64[-1]) / 2.0)
_KCONST = [float(np.exp((_C0 if s < _H else _C1) - _a64[s]))
           for s in range(_BUMP_STEPS)]


def _ect_kernel(xt_ref, b_ref, vt_ref, out_ref):
    i = pl.program_id(0)

    @pl.when(i == 0)
    def _init():
        out_ref[...] = jnp.zeros_like(out_ref)

    nh = jnp.dot(vt_ref[...], xt_ref[...],
                 preferred_element_type=jnp.float32)
    e0 = jnp.exp(jnp.clip(nh - _C0, -88.0, 88.0)).astype(jnp.bfloat16)
    e1 = jnp.exp(jnp.clip(nh - _C1, -88.0, 88.0)).astype(jnp.bfloat16)
    one = jnp.bfloat16(1.0)
    blocks = []
    for s in range(_BUMP_STEPS):
        e = e0 if s < _H else e1
        m1 = one + e * jnp.bfloat16(_KCONST[s])
        blocks.append(one / m1)
    sig = jnp.concatenate(blocks, axis=0)
    seg = b_ref[...]
    oh = (seg == jax.lax.broadcasted_iota(jnp.int32, (1, _NUM_SEGMENTS), 1)
          ).astype(jnp.bfloat16)
    out_ref[...] += jnp.dot(sig, oh,
                            preferred_element_type=jnp.float32)


@jax.jit
def kernel(x, batch, v):
    n = x.shape[0]
    st = _BUMP_STEPS * _NUM_THETAS
    grid = _N_PAD // _CHUNK
    xt = jnp.pad(x, ((0, _N_PAD - n), (0, 0))).T
    batch2d = jnp.pad(batch, (0, _N_PAD - n),
                      constant_values=_NUM_SEGMENTS).reshape(_N_PAD, 1)
    vt = (_SCALE * v.T).astype(jnp.float32)
    out = pl.pallas_call(
        _ect_kernel,
        grid=(grid,),
        in_specs=[
            pl.BlockSpec((x.shape[1], _CHUNK), lambda i: (0, i)),
            pl.BlockSpec((_CHUNK, 1), lambda i: (i, 0)),
            pl.BlockSpec((_NUM_THETAS, v.shape[0]), lambda i: (0, 0)),
        ],
        out_specs=pl.BlockSpec((st, _NUM_SEGMENTS), lambda i: (0, 0)),
        out_shape=jax.ShapeDtypeStruct((st, _NUM_SEGMENTS), jnp.float32),
    )(xt, batch2d, vt)
    return out.T.reshape(_NUM_SEGMENTS, _BUMP_STEPS, _NUM_THETAS)
